# baseline (speedup 1.0000x reference)
import jax
import jax.numpy as jnp
from jax import lax
from jax.experimental import pallas as pl
from jax.experimental.pallas import tpu as pltpu
from jax.experimental.pallas import tpu_sc as plsc

VOCAB_SIZE = 1000000
EMB_SIZE = 32
N = 16384

NUM_CORES = 2
NUM_SUBCORES = 16
NUM_WORKERS = NUM_CORES * NUM_SUBCORES
ELEMS = N * EMB_SIZE
ELEMS_PER_WORKER = ELEMS // NUM_WORKERS


def _gather_body(table_hbm, idx_hbm, out_hbm, idx_v, vals_v, sem):
    wid = lax.axis_index("s") * NUM_CORES + lax.axis_index("c")
    base = wid * ELEMS_PER_WORKER
    pltpu.sync_copy(idx_hbm.at[pl.ds(base, ELEMS_PER_WORKER)], idx_v)
    pltpu.async_copy(table_hbm.at[idx_v], vals_v, sem).wait()
    pltpu.sync_copy(vals_v, out_hbm.at[pl.ds(base, ELEMS_PER_WORKER)])


def _node_gather(table_flat, idx_flat):
    mesh = plsc.VectorSubcoreMesh(
        core_axis_name="c", subcore_axis_name="s",
        num_cores=NUM_CORES, num_subcores=NUM_SUBCORES,
    )
    return pl.kernel(
        _gather_body,
        out_type=jax.ShapeDtypeStruct((ELEMS,), jnp.float32),
        mesh=mesh,
        scratch_types=[
            pltpu.VMEM((ELEMS_PER_WORKER,), jnp.int32),
            pltpu.VMEM((ELEMS_PER_WORKER,), jnp.float32),
            pltpu.SemaphoreType.DMA,
        ],
    )(table_flat, idx_flat)


@jax.jit
def _impl(vocab_ids, selector_ids, node_table, sel_table):
    vidx = vocab_ids.astype(jnp.int32)
    idx_flat = (vidx[:, None] * EMB_SIZE
                + lax.iota(jnp.int32, EMB_SIZE)[None, :]).reshape(ELEMS)
    flat = node_table.reshape(VOCAB_SIZE * EMB_SIZE)
    nodes = _node_gather(flat, idx_flat).reshape(N, EMB_SIZE)
    sel = jnp.take(sel_table, selector_ids.astype(jnp.int32), axis=0)
    return jnp.concatenate([nodes, sel], axis=1)


def kernel(vocab_ids, selector_ids, node_table, sel_table):
    return _impl(vocab_ids, selector_ids, node_table, sel_table)
